# baseline (speedup 1.0000x reference)
import jax
import jax.numpy as jnp
from jax.experimental import pallas as pl

_B, _T, _D1, _D2 = 16, 4096, 64, 4
_K = 4
_DB = _D1 // _K


def _row_kernel(det_ref, y_ref, o_ref):
    tt = jax.lax.broadcasted_iota(jnp.int32, (1, _T), 1)
    m = jnp.isnan(det_ref[0, 0, 0:1, :])
    first = jnp.min(jnp.where(m, tt, _T))
    last = jnp.max(jnp.where(m, tt, -1))
    s = first - 1
    e = last + 1
    base_s = pl.multiple_of((s // 128) * 128, 128)
    base_e = pl.multiple_of((e // 128) * 128, 128)
    y = y_ref[0]
    cs = y_ref[0, :, :, pl.ds(base_s, 128)]
    ce = y_ref[0, :, :, pl.ds(base_e, 128)]
    lane = jax.lax.broadcasted_iota(jnp.int32, (1, 1, 128), 2)
    a = jnp.sum(jnp.where(lane == s - base_s, cs, 0.0), axis=2, keepdims=True)
    b = jnp.sum(jnp.where(lane == e - base_e, ce, 0.0), axis=2, keepdims=True)
    tt3 = jax.lax.broadcasted_iota(jnp.int32, (1, 1, _T), 2)
    step = 1.0 / (e - s).astype(jnp.float32)
    w = (tt3 - s).astype(jnp.float32) * step
    in_gap = (tt3 > s) & (tt3 < e)
    vals = a + w * (b - a)
    o_ref[0] = jnp.where(in_gap, vals, y)


def kernel(x):
    y = jnp.transpose(x, (0, 2, 3, 1))
    out = pl.pallas_call(
        _row_kernel,
        grid=(_B, _K),
        in_specs=[
            pl.BlockSpec((1, 1, _D2, _T), lambda i, k: (i, 0, 0, 0)),
            pl.BlockSpec((1, _DB, _D2, _T), lambda i, k: (i, k, 0, 0)),
        ],
        out_specs=pl.BlockSpec((1, _DB, _D2, _T), lambda i, k: (i, k, 0, 0)),
        out_shape=jax.ShapeDtypeStruct((_B, _D1, _D2, _T), jnp.float32),
    )(y, y)
    return jnp.transpose(out, (0, 3, 1, 2))


# speedup vs baseline: 1.8941x; 1.8941x over previous
import jax
import jax.numpy as jnp
from jax.experimental import pallas as pl
from jax.experimental.pallas import tpu as pltpu

_B, _T, _D1, _D2 = 16, 4096, 64, 4
_K = 2
_DB = _D1 // _K
_N = 6
_NC = _B * _K
_W = 1152


def _in_copy(x_hbm, bufs, in_sems, idx, j):
    r, k = idx // _K, (idx % _K) * _DB
    return pltpu.make_async_copy(
        x_hbm.at[r, pl.ds(k, _DB)], bufs.at[j], in_sems.at[j])


def _out_copy(o_hbm, obufs, out_sems, idx, j):
    r, k = idx // _K, (idx % _K) * _DB
    return pltpu.make_async_copy(
        obufs.at[j], o_hbm.at[r, pl.ds(k, _DB)], out_sems.at[j])


def _main(x_hbm, o_hbm, det, bufs, obufs, s_ref, e_ref,
          det_sem, in_sems, out_sems):
    dcp = pltpu.make_async_copy(x_hbm.at[:, 0:1], det, det_sem)
    dcp.start()
    for j in range(_N):
        _in_copy(x_hbm, bufs, in_sems, j, j).start()
    dcp.wait()

    def det_body(r, carry):
        tt = jax.lax.broadcasted_iota(jnp.int32, (1, _T), 1)
        m = jnp.isnan(det[r, 0, 0:1, :])
        first = jnp.min(jnp.where(m, tt, _T))
        last = jnp.max(jnp.where(m, tt, -1))
        s_ref[r] = first - 1
        e_ref[r] = last + 1
        return carry
    jax.lax.fori_loop(0, _B, det_body, 0)

    def body(idx, carry):
        j = idx % _N
        r = idx // _K

        @pl.when(idx >= _N)
        def _():
            _out_copy(o_hbm, obufs, out_sems, idx - _N, j).wait()

        _in_copy(x_hbm, bufs, in_sems, idx, j).wait()

        s, e = s_ref[r], e_ref[r]
        step = 1.0 / (e - s).astype(jnp.float32)
        obufs[j] = bufs[j]
        base = pl.multiple_of((s // 128) * 128, 128)
        gap_fits = ((e - base) < _W) & (base + _W <= _T)

        @pl.when(gap_fits)
        def _():
            win = bufs[j, :, :, pl.ds(base, _W)]
            lane = base + jax.lax.broadcasted_iota(jnp.int32, (1, 1, _W), 2)
            a = jnp.sum(jnp.where(lane[:, :, 0:128] == s,
                                  win[:, :, 0:128], 0.0),
                        axis=2, keepdims=True)
            b = jnp.sum(jnp.where(lane == e, win, 0.0),
                        axis=2, keepdims=True)
            w = (lane - s).astype(jnp.float32) * step
            vals = a + w * (b - a)
            in_gap = (lane > s) & (lane < e)
            obufs[j, :, :, pl.ds(base, _W)] = jnp.where(in_gap, vals, win)

        @pl.when(jnp.logical_not(gap_fits))
        def _():
            y = bufs[j]
            tt3 = jax.lax.broadcasted_iota(jnp.int32, (1, 1, _T), 2)
            base_e = pl.multiple_of((e // 128) * 128, 128)
            cs = bufs[j, :, :, pl.ds(base, 128)]
            ce = bufs[j, :, :, pl.ds(base_e, 128)]
            lane = jax.lax.broadcasted_iota(jnp.int32, (1, 1, 128), 2)
            a = jnp.sum(jnp.where(lane == s - base, cs, 0.0),
                        axis=2, keepdims=True)
            b = jnp.sum(jnp.where(lane == e - base_e, ce, 0.0),
                        axis=2, keepdims=True)
            w = (tt3 - s).astype(jnp.float32) * step
            vals = a + w * (b - a)
            in_gap = (tt3 > s) & (tt3 < e)
            obufs[j] = jnp.where(in_gap, vals, y)

        _out_copy(o_hbm, obufs, out_sems, idx, j).start()

        @pl.when(idx + _N < _NC)
        def _():
            _in_copy(x_hbm, bufs, in_sems, idx + _N, j).start()
        return carry
    jax.lax.fori_loop(0, _NC, body, 0)

    for idx in range(_NC - _N, _NC):
        _out_copy(o_hbm, obufs, out_sems, idx, idx % _N).wait()


def kernel(x):
    y = jnp.transpose(x, (0, 2, 3, 1))
    out = pl.pallas_call(
        _main,
        in_specs=[pl.BlockSpec(memory_space=pltpu.HBM)],
        out_specs=pl.BlockSpec(memory_space=pltpu.HBM),
        out_shape=jax.ShapeDtypeStruct((_B, _D1, _D2, _T), jnp.float32),
        scratch_shapes=[
            pltpu.VMEM((_B, 1, _D2, _T), jnp.float32),
            pltpu.VMEM((_N, _DB, _D2, _T), jnp.float32),
            pltpu.VMEM((_N, _DB, _D2, _T), jnp.float32),
            pltpu.SMEM((_B,), jnp.int32),
            pltpu.SMEM((_B,), jnp.int32),
            pltpu.SemaphoreType.DMA,
            pltpu.SemaphoreType.DMA((_N,)),
            pltpu.SemaphoreType.DMA((_N,)),
        ],
    )(y)
    return jnp.transpose(out, (0, 3, 1, 2))


# speedup vs baseline: 1.8989x; 1.0025x over previous
import jax
import jax.numpy as jnp
from jax.experimental import pallas as pl
from jax.experimental.pallas import tpu as pltpu

_B, _T, _D1, _D2 = 16, 4096, 64, 4
_K = 2
_DB = _D1 // _K
_N = 8
_NC = _B * _K
_W = 1152


def _in_copy(x_hbm, bufs, in_sems, idx, j):
    r, k = idx // _K, (idx % _K) * _DB
    return pltpu.make_async_copy(
        x_hbm.at[r, pl.ds(k, _DB)], bufs.at[j], in_sems.at[j])


def _out_copy(o_hbm, obufs, out_sems, idx, j):
    r, k = idx // _K, (idx % _K) * _DB
    return pltpu.make_async_copy(
        obufs.at[j], o_hbm.at[r, pl.ds(k, _DB)], out_sems.at[j])


def _main(x_hbm, o_hbm, det, bufs, obufs, s_ref, e_ref,
          det_sem, in_sems, out_sems):
    dcp = pltpu.make_async_copy(x_hbm.at[:, 0:1], det, det_sem)
    dcp.start()
    for j in range(_N):
        _in_copy(x_hbm, bufs, in_sems, j, j).start()
    dcp.wait()

    def det_body(r, carry):
        tt = jax.lax.broadcasted_iota(jnp.int32, (1, _T), 1)
        m = jnp.isnan(det[r, 0, 0:1, :])
        first = jnp.min(jnp.where(m, tt, _T))
        last = jnp.max(jnp.where(m, tt, -1))
        s_ref[r] = first - 1
        e_ref[r] = last + 1
        return carry
    jax.lax.fori_loop(0, _B, det_body, 0)

    def body(idx, carry):
        j = idx % _N
        r = idx // _K

        @pl.when(idx >= _N)
        def _():
            _out_copy(o_hbm, obufs, out_sems, idx - _N, j).wait()

        _in_copy(x_hbm, bufs, in_sems, idx, j).wait()

        s, e = s_ref[r], e_ref[r]
        step = 1.0 / (e - s).astype(jnp.float32)
        obufs[j] = bufs[j]
        base = pl.multiple_of((s // 128) * 128, 128)
        gap_fits = ((e - base) < _W) & (base + _W <= _T)

        @pl.when(gap_fits)
        def _():
            win = bufs[j, :, :, pl.ds(base, _W)]
            lane = base + jax.lax.broadcasted_iota(jnp.int32, (1, 1, _W), 2)
            a = jnp.sum(jnp.where(lane[:, :, 0:128] == s,
                                  win[:, :, 0:128], 0.0),
                        axis=2, keepdims=True)
            b = jnp.sum(jnp.where(lane == e, win, 0.0),
                        axis=2, keepdims=True)
            w = (lane - s).astype(jnp.float32) * step
            vals = a + w * (b - a)
            in_gap = (lane > s) & (lane < e)
            obufs[j, :, :, pl.ds(base, _W)] = jnp.where(in_gap, vals, win)

        @pl.when(jnp.logical_not(gap_fits))
        def _():
            y = bufs[j]
            tt3 = jax.lax.broadcasted_iota(jnp.int32, (1, 1, _T), 2)
            base_e = pl.multiple_of((e // 128) * 128, 128)
            cs = bufs[j, :, :, pl.ds(base, 128)]
            ce = bufs[j, :, :, pl.ds(base_e, 128)]
            lane = jax.lax.broadcasted_iota(jnp.int32, (1, 1, 128), 2)
            a = jnp.sum(jnp.where(lane == s - base, cs, 0.0),
                        axis=2, keepdims=True)
            b = jnp.sum(jnp.where(lane == e - base_e, ce, 0.0),
                        axis=2, keepdims=True)
            w = (tt3 - s).astype(jnp.float32) * step
            vals = a + w * (b - a)
            in_gap = (tt3 > s) & (tt3 < e)
            obufs[j] = jnp.where(in_gap, vals, y)

        _out_copy(o_hbm, obufs, out_sems, idx, j).start()

        @pl.when(idx + _N < _NC)
        def _():
            _in_copy(x_hbm, bufs, in_sems, idx + _N, j).start()
        return carry
    jax.lax.fori_loop(0, _NC, body, 0)

    for idx in range(_NC - _N, _NC):
        _out_copy(o_hbm, obufs, out_sems, idx, idx % _N).wait()


def kernel(x):
    y = jnp.transpose(x, (0, 2, 3, 1))
    out = pl.pallas_call(
        _main,
        in_specs=[pl.BlockSpec(memory_space=pltpu.HBM)],
        out_specs=pl.BlockSpec(memory_space=pltpu.HBM),
        out_shape=jax.ShapeDtypeStruct((_B, _D1, _D2, _T), jnp.float32),
        scratch_shapes=[
            pltpu.VMEM((_B, 1, _D2, _T), jnp.float32),
            pltpu.VMEM((_N, _DB, _D2, _T), jnp.float32),
            pltpu.VMEM((_N, _DB, _D2, _T), jnp.float32),
            pltpu.SMEM((_B,), jnp.int32),
            pltpu.SMEM((_B,), jnp.int32),
            pltpu.SemaphoreType.DMA,
            pltpu.SemaphoreType.DMA((_N,)),
            pltpu.SemaphoreType.DMA((_N,)),
        ],
    )(y)
    return jnp.transpose(out, (0, 3, 1, 2))
